# baseline (speedup 1.0000x reference)
import jax
import jax.numpy as jnp
from jax import lax
from jax.experimental import pallas as pl
from jax.experimental.pallas import tpu as pltpu
from jax.experimental.pallas import tpu_sc as plsc

_NUM_FEATURES = 100
_EMBED_DIM = 32
_BATCH = 16384
_ROW = _NUM_FEATURES * _EMBED_DIM
_LANES = 16
_VECS = _ROW // _LANES

_NC = 2
_NS = 16
_NW = _NC * _NS
_RPW = _BATCH // _NW
_C = 16
_NCH = _RPW // _C


def _sc_body(x_hbm, tab_hbm, out_hbm, tab_v, buf):
    wid = lax.axis_index("s") * _NC + lax.axis_index("c")
    base = wid * _RPW
    pltpu.sync_copy(tab_hbm, tab_v)

    def chunk(i, carry):
        r0 = base + i * _C
        pltpu.sync_copy(x_hbm.at[pl.ds(r0, _C)], buf)

        def jloop(j, c2):
            j16 = j * _LANES
            t = tab_v[pl.ds(j16, _LANES)]
            for r in range(_C):
                buf[r, pl.ds(j16, _LANES)] = buf[r, pl.ds(j16, _LANES)] + t
            return c2

        lax.fori_loop(0, _VECS, jloop, 0)
        pltpu.sync_copy(buf, out_hbm.at[pl.ds(r0, _C)])
        return carry

    lax.fori_loop(0, _NCH, chunk, 0)


def kernel(inputs, column_table):
    x = inputs.reshape(_BATCH, _ROW)
    tab = column_table.reshape(_ROW)
    mesh = plsc.VectorSubcoreMesh(core_axis_name="c", subcore_axis_name="s")
    out = pl.kernel(
        _sc_body,
        out_type=jax.ShapeDtypeStruct((_BATCH, _ROW), jnp.float32),
        mesh=mesh,
        scratch_types=[
            pltpu.VMEM((_ROW,), jnp.float32),
            pltpu.VMEM((_C, _ROW), jnp.float32),
        ],
    )(x, tab)
    return out.reshape(_BATCH, _NUM_FEATURES, _EMBED_DIM)
